# baseline (device time: 113394 ns/iter reference)
import jax
import jax.numpy as jnp
from jax import lax
from jax.experimental import pallas as pl
from jax.experimental.pallas import tpu as pltpu

B = 32
NB = 256
BS = 32
H = 16
D = 128
P_GLOBAL = 512
P_LOCAL = 256
PY = P_LOCAL // 2
CP = 32
NC = PY // CP
CK = CP * BS
MYKEYS = PY * BS
NEG = -1e9
SCALE = D ** -0.5


def kernel(Q, K, V, bt, lens):
    lens2 = lens.reshape(B, 1)

    def body(q_ref, k_hbm, v_hbm, bt_ref, lens_ref, out_ref,
             lw_ref, oacc, mlacc, o_rcv, ml_rcv, o_rcv2, ml_rcv2,
             kbuf, vbuf, kv_sems, send_sems, recv_sems):
        c = pl.program_id(0)
        my_x = lax.axis_index("x")
        my_y = lax.axis_index("y")
        page0 = my_y * PY
        slot = c % 2
        nxt = (c + 1) % 2

        def stage_start(chunk, sl):
            pltpu.make_async_copy(
                k_hbm.at[pl.ds(page0 + chunk * CP, CP)],
                kbuf.at[sl], kv_sems.at[0, sl]).start()
            pltpu.make_async_copy(
                v_hbm.at[pl.ds(page0 + chunk * CP, CP)],
                vbuf.at[sl], kv_sems.at[1, sl]).start()

        def stage_wait(chunk, sl):
            pltpu.make_async_copy(
                k_hbm.at[pl.ds(page0 + chunk * CP, CP)],
                kbuf.at[sl], kv_sems.at[0, sl]).wait()
            pltpu.make_async_copy(
                v_hbm.at[pl.ds(page0 + chunk * CP, CP)],
                vbuf.at[sl], kv_sems.at[1, sl]).wait()

        @pl.when(c == 0)
        def _():
            stage_start(0, 0)

            barrier_sem = pltpu.get_barrier_semaphore()
            pl.semaphore_signal(
                barrier_sem, inc=1, device_id=(1 - my_x, my_y),
                device_id_type=pl.DeviceIdType.MESH)
            pl.semaphore_signal(
                barrier_sem, inc=1, device_id=(my_x, 1 - my_y),
                device_id_type=pl.DeviceIdType.MESH)
            pl.semaphore_wait(barrier_sem, 2)

            gbase = my_x * P_LOCAL + page0
            page_ids = (
                lax.broadcasted_iota(jnp.int32, (1, 1, PY), 2) + gbase
            )
            slot_ids = lax.broadcasted_iota(jnp.int32, (1, NB, 1), 1)
            hits = (bt_ref[...][:, :, None] == page_ids) & (
                slot_ids < lens_ref[...][:, :, None]
            )
            W = jnp.sum(hits.astype(jnp.float32), axis=1)
            logw = jnp.where(W > 0, jnp.log(jnp.maximum(W, 1e-37)), NEG)
            erow = lax.broadcasted_iota(jnp.int32, (PY, MYKEYS), 0)
            ecol = lax.broadcasted_iota(jnp.int32, (PY, MYKEYS), 1)
            E = (ecol // BS == erow).astype(jnp.bfloat16)
            lw_ref[...] = lax.dot_general(
                logw.astype(jnp.bfloat16), E, (((1,), (0,)), ((), ())),
                preferred_element_type=jnp.float32,
            )

            mlacc[0] = jnp.full((H, B, 1), NEG, jnp.float32)
            mlacc[1] = jnp.zeros((H, B, 1), jnp.float32)
            oacc[...] = jnp.zeros((H, B, D), jnp.float32)

        @pl.when(c + 1 < NC)
        def _():
            stage_start(c + 1, nxt)

        stage_wait(c, slot)

        for h in range(H):
            q = q_ref[:, 0, h, :].astype(jnp.bfloat16)
            k = kbuf[slot][:, :, h, :].reshape(CK, D).astype(jnp.bfloat16)
            v = vbuf[slot][:, :, h, :].reshape(CK, D).astype(jnp.bfloat16)
            s = lax.dot_general(
                q, k, (((1,), (1,)), ((), ())),
                preferred_element_type=jnp.float32,
            )
            s = s * SCALE + lw_ref[:, pl.ds(c * CK, CK)]
            mc = jnp.max(s, axis=1, keepdims=True)
            m_old = mlacc[0, h]
            m_new = jnp.maximum(m_old, mc)
            p = jnp.where(m_new > -1e8, jnp.exp(s - m_new), 0.0)
            alpha = jnp.exp(m_old - m_new)
            mlacc[0, h] = m_new
            mlacc[1, h] = alpha * mlacc[1, h] + jnp.sum(p, axis=1,
                                                        keepdims=True)
            o = lax.dot_general(
                p.astype(jnp.bfloat16), v, (((1,), (0,)), ((), ())),
                preferred_element_type=jnp.float32,
            )
            oacc[h] = alpha * oacc[h] + o

        @pl.when(c == NC - 1)
        def _():
            def exchange(o_dst, ml_dst, peer, s0):
                ro = pltpu.make_async_remote_copy(
                    src_ref=oacc, dst_ref=o_dst,
                    send_sem=send_sems.at[s0], recv_sem=recv_sems.at[s0],
                    device_id=peer, device_id_type=pl.DeviceIdType.MESH)
                rml = pltpu.make_async_remote_copy(
                    src_ref=mlacc, dst_ref=ml_dst,
                    send_sem=send_sems.at[s0 + 1],
                    recv_sem=recv_sems.at[s0 + 1],
                    device_id=peer, device_id_type=pl.DeviceIdType.MESH)
                ro.start()
                rml.start()
                ro.wait()
                rml.wait()

            def merge(o_b, ml_b):
                ma, la = mlacc[0], mlacc[1]
                mb, lb = ml_b[0], ml_b[1]
                mm = jnp.maximum(ma, mb)
                ea = jnp.exp(ma - mm)
                eb = jnp.exp(mb - mm)
                lt = la * ea + lb * eb
                ot = oacc[...] * ea + o_b[...] * eb
                return mm, lt, ot

            exchange(o_rcv, ml_rcv, (1 - my_x, my_y), 0)
            mm, lt, ot = merge(o_rcv, ml_rcv)
            mlacc[0] = mm
            mlacc[1] = lt
            oacc[...] = ot

            exchange(o_rcv2, ml_rcv2, (my_x, 1 - my_y), 2)
            _, lt2, ot2 = merge(o_rcv2, ml_rcv2)
            inv = 1.0 / lt2
            for hh in range(H):
                out_ref[:, hh * D:(hh + 1) * D] = ot2[hh] * inv[hh]

    out2 = pl.pallas_call(
        body,
        grid=(NC,),
        in_specs=[
            pl.BlockSpec((B, 1, H, D), lambda c: (0, 0, 0, 0)),
            pl.BlockSpec(memory_space=pltpu.MemorySpace.HBM),
            pl.BlockSpec(memory_space=pltpu.MemorySpace.HBM),
            pl.BlockSpec((B, NB), lambda c: (0, 0)),
            pl.BlockSpec((B, 1), lambda c: (0, 0)),
        ],
        out_specs=pl.BlockSpec((B, H * D), lambda c: (0, 0)),
        out_shape=jax.ShapeDtypeStruct((B, H * D), jnp.float32),
        scratch_shapes=[
            pltpu.VMEM((B, MYKEYS), jnp.float32),
            pltpu.VMEM((H, B, D), jnp.float32),
            pltpu.VMEM((2, H, B, 1), jnp.float32),
            pltpu.VMEM((H, B, D), jnp.float32),
            pltpu.VMEM((2, H, B, 1), jnp.float32),
            pltpu.VMEM((H, B, D), jnp.float32),
            pltpu.VMEM((2, H, B, 1), jnp.float32),
            pltpu.VMEM((2, CP, BS, H, D), jnp.float32),
            pltpu.VMEM((2, CP, BS, H, D), jnp.float32),
            pltpu.SemaphoreType.DMA((2, 2)),
            pltpu.SemaphoreType.DMA((4,)),
            pltpu.SemaphoreType.DMA((4,)),
        ],
        compiler_params=pltpu.CompilerParams(
            dimension_semantics=("arbitrary",),
            collective_id=0,
            vmem_limit_bytes=60 * 1024 * 1024,
        ),
    )(Q, K, V, bt, lens2)
    return out2.reshape(B, 1, H, D)


# device time: 39479 ns/iter; 2.8723x vs baseline; 2.8723x over previous
import jax
import jax.numpy as jnp
from jax import lax
from jax.experimental import pallas as pl
from jax.experimental.pallas import tpu as pltpu

B = 32
NB = 256
BS = 32
H = 16
HL = H // 2
HD = HL * 128
D = 128
P_GLOBAL = 512
P_LOCAL = 256
KEYS = P_LOCAL * BS
CP = 64
NC = P_LOCAL // CP
CK = CP * BS
NEG = -1e9
SCALE = D ** -0.5
SUB = 2
PP = CP // SUB


def kernel(Q, K, V, bt, lens):
    lens2 = lens.reshape(B, 1)

    def body(q_ref, k_hbm, v_hbm, bt_ref, lens_ref, out_ref,
             lw_ref, oacc, lacc, o_rcv, l_rcv,
             kbuf, vbuf, kv_sems, send_sems, recv_sems):
        c = pl.program_id(0)
        my_x = lax.axis_index("x")
        my_y = lax.axis_index("y")
        hbase = my_y * HD
        slot = c % 2
        nxt = (c + 1) % 2

        k3 = k_hbm.reshape(P_LOCAL, BS, H * D)
        v3 = v_hbm.reshape(P_LOCAL, BS, H * D)

        def stage_start(chunk, sl):
            for i in range(SUB):
                pltpu.make_async_copy(
                    k3.at[pl.ds(chunk * CP + i * PP, PP), :,
                          pl.ds(hbase, HD)],
                    kbuf.at[sl, pl.ds(i * PP, PP)],
                    kv_sems.at[0, sl, i]).start()
                pltpu.make_async_copy(
                    v3.at[pl.ds(chunk * CP + i * PP, PP), :,
                          pl.ds(hbase, HD)],
                    vbuf.at[sl, pl.ds(i * PP, PP)],
                    kv_sems.at[1, sl, i]).start()

        def stage_wait(chunk, sl):
            for i in range(SUB):
                pltpu.make_async_copy(
                    k3.at[pl.ds(chunk * CP + i * PP, PP), :,
                          pl.ds(hbase, HD)],
                    kbuf.at[sl, pl.ds(i * PP, PP)],
                    kv_sems.at[0, sl, i]).wait()
                pltpu.make_async_copy(
                    v3.at[pl.ds(chunk * CP + i * PP, PP), :,
                          pl.ds(hbase, HD)],
                    vbuf.at[sl, pl.ds(i * PP, PP)],
                    kv_sems.at[1, sl, i]).wait()

        @pl.when(c == 0)
        def _():
            stage_start(0, 0)

            barrier_sem = pltpu.get_barrier_semaphore()
            pl.semaphore_signal(
                barrier_sem, inc=1, device_id=(1 - my_x, my_y),
                device_id_type=pl.DeviceIdType.MESH)
            pl.semaphore_signal(
                barrier_sem, inc=1, device_id=(my_x, 1 - my_y),
                device_id_type=pl.DeviceIdType.MESH)
            pl.semaphore_wait(barrier_sem, 2)

            page_ids = (
                lax.broadcasted_iota(jnp.int32, (1, 1, P_LOCAL), 2)
                + my_x * P_LOCAL
            )
            slot_ids = lax.broadcasted_iota(jnp.int32, (1, NB, 1), 1)
            hits = (bt_ref[...][:, :, None] == page_ids) & (
                slot_ids < lens_ref[...][:, :, None]
            )
            W = jnp.sum(hits.astype(jnp.float32), axis=1)
            logw = jnp.where(W > 0, jnp.log(jnp.maximum(W, 1e-37)), NEG)
            erow = lax.broadcasted_iota(jnp.int32, (P_LOCAL, KEYS), 0)
            ecol = lax.broadcasted_iota(jnp.int32, (P_LOCAL, KEYS), 1)
            E = (ecol // BS == erow).astype(jnp.bfloat16)
            lw_ref[...] = lax.dot_general(
                logw.astype(jnp.bfloat16), E, (((1,), (0,)), ((), ())),
                preferred_element_type=jnp.float32,
            )

            lacc[...] = jnp.zeros((HL, B, 1), jnp.float32)
            oacc[...] = jnp.zeros((HL, B, D), jnp.float32)

        @pl.when(c + 1 < NC)
        def _():
            stage_start(c + 1, nxt)

        stage_wait(c, slot)

        for h in range(HL):
            q = q_ref[:, 0, my_y * HL + h, :] * SCALE
            qb = q.astype(jnp.bfloat16)
            k = kbuf[slot][:, :, h * D:(h + 1) * D].reshape(CK, D)
            v = vbuf[slot][:, :, h * D:(h + 1) * D].reshape(CK, D)
            s = lax.dot_general(
                qb, k.astype(jnp.bfloat16), (((1,), (1,)), ((), ())),
                preferred_element_type=jnp.float32,
            )
            p = jnp.exp(s + lw_ref[:, pl.ds(c * CK, CK)])
            lacc[h] = lacc[h] + jnp.sum(p, axis=1, keepdims=True)
            o = lax.dot_general(
                p.astype(jnp.bfloat16), v.astype(jnp.bfloat16),
                (((1,), (0,)), ((), ())),
                preferred_element_type=jnp.float32,
            )
            oacc[h] = oacc[h] + o

        @pl.when(c == NC - 1)
        def _():
            peer_x = (1 - my_x, my_y)
            rdma_o = pltpu.make_async_remote_copy(
                src_ref=oacc, dst_ref=o_rcv,
                send_sem=send_sems.at[0], recv_sem=recv_sems.at[0],
                device_id=peer_x, device_id_type=pl.DeviceIdType.MESH,
            )
            rdma_l = pltpu.make_async_remote_copy(
                src_ref=lacc, dst_ref=l_rcv,
                send_sem=send_sems.at[1], recv_sem=recv_sems.at[1],
                device_id=peer_x, device_id_type=pl.DeviceIdType.MESH,
            )
            rdma_o.start()
            rdma_l.start()
            rdma_o.wait()
            rdma_l.wait()

            lt = lacc[...] + l_rcv[...]
            ot = oacc[...] + o_rcv[...]
            inv = 1.0 / lt
            base = my_y * HD
            for hh in range(HL):
                out_ref[:, pl.ds(base + hh * D, D)] = ot[hh] * inv[hh]

            rdma_y = pltpu.make_async_remote_copy(
                src_ref=out_ref.at[:, pl.ds(base, HD)],
                dst_ref=out_ref.at[:, pl.ds(base, HD)],
                send_sem=send_sems.at[2], recv_sem=recv_sems.at[2],
                device_id=(my_x, 1 - my_y),
                device_id_type=pl.DeviceIdType.MESH,
            )
            rdma_y.start()
            rdma_y.wait()

    out2 = pl.pallas_call(
        body,
        grid=(NC,),
        in_specs=[
            pl.BlockSpec((B, 1, H, D), lambda c: (0, 0, 0, 0)),
            pl.BlockSpec(memory_space=pltpu.MemorySpace.HBM),
            pl.BlockSpec(memory_space=pltpu.MemorySpace.HBM),
            pl.BlockSpec((B, NB), lambda c: (0, 0)),
            pl.BlockSpec((B, 1), lambda c: (0, 0)),
        ],
        out_specs=pl.BlockSpec((B, H * D), lambda c: (0, 0)),
        out_shape=jax.ShapeDtypeStruct((B, H * D), jnp.float32),
        scratch_shapes=[
            pltpu.VMEM((B, KEYS), jnp.float32),
            pltpu.VMEM((HL, B, D), jnp.float32),
            pltpu.VMEM((HL, B, 1), jnp.float32),
            pltpu.VMEM((HL, B, D), jnp.float32),
            pltpu.VMEM((HL, B, 1), jnp.float32),
            pltpu.VMEM((2, CP, BS, HD), jnp.float32),
            pltpu.VMEM((2, CP, BS, HD), jnp.float32),
            pltpu.SemaphoreType.DMA((2, 2, SUB)),
            pltpu.SemaphoreType.DMA((3,)),
            pltpu.SemaphoreType.DMA((3,)),
        ],
        compiler_params=pltpu.CompilerParams(
            dimension_semantics=("arbitrary",),
            collective_id=0,
            vmem_limit_bytes=60 * 1024 * 1024,
        ),
    )(Q, K, V, bt, lens2)
    return out2.reshape(B, 1, H, D)


# device time: 36213 ns/iter; 3.1313x vs baseline; 1.0902x over previous
import jax
import jax.numpy as jnp
from jax import lax
from jax.experimental import pallas as pl
from jax.experimental.pallas import tpu as pltpu

B = 32
NB = 256
BS = 32
H = 16
HL = H // 2
D = 128
P_GLOBAL = 512
P_LOCAL = 256
KEYS = P_LOCAL * BS
NEG = -1e9
SCALE = D ** -0.5
SUB = 4
PP = P_LOCAL // SUB


def kernel(Q, K, V, bt, lens):
    lens2 = lens.reshape(B, 1)

    def _stage_start(hbm, buf, sems, head, slot):
        for i in range(SUB):
            pltpu.make_async_copy(
                hbm.at[pl.ds(i * PP, PP), :, head, :],
                buf.at[slot, pl.ds(i * PP, PP)],
                sems.at[slot, i],
            ).start()

    def _stage_wait(hbm, buf, sems, head, slot):
        for i in range(SUB):
            pltpu.make_async_copy(
                hbm.at[pl.ds(i * PP, PP), :, head, :],
                buf.at[slot, pl.ds(i * PP, PP)],
                sems.at[slot, i],
            ).wait()

    def body(q_ref, k_hbm, v_hbm, bt_ref, lens_ref, out_ref,
             lw_ref, o_acc, l_acc, o_rcv, l_rcv,
             kbuf, vbuf, k_sems, v_sems, ox_sems, send_sems, recv_sems):
        h = pl.program_id(0)
        my_y = lax.axis_index("y")
        head = my_y * HL + h
        slot = h % 2
        nxt = (h + 1) % 2

        @pl.when(h == 0)
        def _():
            _stage_start(k_hbm, kbuf, k_sems, head, 0)
            _stage_start(v_hbm, vbuf, v_sems, head, 0)

            my_x = lax.axis_index("x")
            barrier_sem = pltpu.get_barrier_semaphore()
            pl.semaphore_signal(
                barrier_sem, inc=1, device_id=(1 - my_x, my_y),
                device_id_type=pl.DeviceIdType.MESH)
            pl.semaphore_signal(
                barrier_sem, inc=1, device_id=(my_x, 1 - my_y),
                device_id_type=pl.DeviceIdType.MESH)
            pl.semaphore_wait(barrier_sem, 2)
            page_ids = (
                lax.broadcasted_iota(jnp.int32, (1, 1, P_LOCAL), 2)
                + my_x * P_LOCAL
            )
            slot_ids = lax.broadcasted_iota(jnp.int32, (1, NB, 1), 1)
            hits = (bt_ref[...][:, :, None] == page_ids) & (
                slot_ids < lens_ref[...][:, :, None]
            )
            W = jnp.sum(hits.astype(jnp.float32), axis=1)
            logw = jnp.where(W > 0, jnp.log(jnp.maximum(W, 1e-37)), NEG)
            erow = lax.broadcasted_iota(jnp.int32, (P_LOCAL, KEYS), 0)
            ecol = lax.broadcasted_iota(jnp.int32, (P_LOCAL, KEYS), 1)
            E = (ecol // BS == erow).astype(jnp.bfloat16)
            lw_ref[...] = lax.dot_general(
                logw.astype(jnp.bfloat16), E, (((1,), (0,)), ((), ())),
                preferred_element_type=jnp.float32,
            )

        @pl.when(h + 1 < HL)
        def _():
            _stage_start(k_hbm, kbuf, k_sems, head + 1, nxt)
            _stage_start(v_hbm, vbuf, v_sems, head + 1, nxt)

        _stage_wait(k_hbm, kbuf, k_sems, head, slot)
        _stage_wait(v_hbm, vbuf, v_sems, head, slot)

        my_x = lax.axis_index("x")
        peer_x = (1 - my_x, my_y)

        q = (q_ref[:, 0, head, :] * SCALE).astype(jnp.bfloat16)
        k = kbuf[slot].reshape(KEYS, D).astype(jnp.bfloat16)
        v = vbuf[slot].reshape(KEYS, D).astype(jnp.bfloat16)
        s = lax.dot_general(
            q, k, (((1,), (1,)), ((), ())),
            preferred_element_type=jnp.float32,
        )
        p = jnp.exp(s + lw_ref[...])
        l = jnp.sum(p, axis=1, keepdims=True)
        o = lax.dot_general(
            p.astype(jnp.bfloat16), v, (((1,), (0,)), ((), ())),
            preferred_element_type=jnp.float32,
        )
        o_acc[h] = o
        l_acc[h] = l

        rdma_oh = pltpu.make_async_remote_copy(
            src_ref=o_acc.at[h], dst_ref=o_rcv.at[h],
            send_sem=ox_sems.at[0, h], recv_sem=ox_sems.at[1, h],
            device_id=peer_x, device_id_type=pl.DeviceIdType.MESH,
        )
        rdma_oh.start()

        @pl.when(h == HL - 1)
        def _():
            rdma_l = pltpu.make_async_remote_copy(
                src_ref=l_acc, dst_ref=l_rcv,
                send_sem=send_sems.at[0], recv_sem=recv_sems.at[0],
                device_id=peer_x, device_id_type=pl.DeviceIdType.MESH,
            )
            rdma_l.start()
            for hh in range(HL):
                pltpu.make_async_remote_copy(
                    src_ref=o_acc.at[hh], dst_ref=o_rcv.at[hh],
                    send_sem=ox_sems.at[0, hh], recv_sem=ox_sems.at[1, hh],
                    device_id=peer_x, device_id_type=pl.DeviceIdType.MESH,
                ).wait()
            rdma_l.wait()

            lt = l_acc[...] + l_rcv[...]
            ot = o_acc[...] + o_rcv[...]
            inv = 1.0 / lt
            base = my_y * (HL * D)
            for hh in range(HL):
                out_ref[:, pl.ds(base + hh * D, D)] = ot[hh] * inv[hh]

            rdma_y = pltpu.make_async_remote_copy(
                src_ref=out_ref.at[:, pl.ds(base, HL * D)],
                dst_ref=out_ref.at[:, pl.ds(base, HL * D)],
                send_sem=send_sems.at[1], recv_sem=recv_sems.at[1],
                device_id=(my_x, 1 - my_y),
                device_id_type=pl.DeviceIdType.MESH,
            )
            rdma_y.start()
            rdma_y.wait()

    out2 = pl.pallas_call(
        body,
        grid=(HL,),
        in_specs=[
            pl.BlockSpec((B, 1, H, D), lambda h: (0, 0, 0, 0)),
            pl.BlockSpec(memory_space=pltpu.MemorySpace.HBM),
            pl.BlockSpec(memory_space=pltpu.MemorySpace.HBM),
            pl.BlockSpec((B, NB), lambda h: (0, 0)),
            pl.BlockSpec((B, 1), lambda h: (0, 0)),
        ],
        out_specs=pl.BlockSpec((B, H * D), lambda h: (0, 0)),
        out_shape=jax.ShapeDtypeStruct((B, H * D), jnp.float32),
        scratch_shapes=[
            pltpu.VMEM((B, KEYS), jnp.float32),
            pltpu.VMEM((HL, B, D), jnp.float32),
            pltpu.VMEM((HL, B, 1), jnp.float32),
            pltpu.VMEM((HL, B, D), jnp.float32),
            pltpu.VMEM((HL, B, 1), jnp.float32),
            pltpu.VMEM((2, P_LOCAL, BS, D), jnp.float32),
            pltpu.VMEM((2, P_LOCAL, BS, D), jnp.float32),
            pltpu.SemaphoreType.DMA((2, SUB)),
            pltpu.SemaphoreType.DMA((2, SUB)),
            pltpu.SemaphoreType.DMA((2, HL)),
            pltpu.SemaphoreType.DMA((2,)),
            pltpu.SemaphoreType.DMA((2,)),
        ],
        compiler_params=pltpu.CompilerParams(
            dimension_semantics=("arbitrary",),
            collective_id=0,
        ),
    )(Q, K, V, bt, lens2)
    return out2.reshape(B, 1, H, D)
